# Initial kernel scaffold; baseline (speedup 1.0000x reference)
import jax
import jax.numpy as jnp
from jax.experimental import pallas as pl


def kernel(x, weight):
    raise NotImplementedError("write your pallas kernel here")


# speedup vs baseline: 1.8683x; 1.8683x over previous
import functools

import jax
import jax.numpy as jnp
from jax import lax
from jax.experimental import pallas as pl
from jax.experimental.pallas import tpu as pltpu
from jax.experimental.pallas import tpu_sc as plsc

NUM_EMBEDDINGS = 1000000
D = 64
BATCH = 16384
HIST = 50
B_TOTAL = BATCH * HIST

NC = 2
NS = 16
NW = NC * NS
G = 128
PER_W = B_TOTAL // NW
NG = PER_W // G
NBUF = 4


def _emb_body(idx_hbm, table_hbm, out_hbm, idx_v, b0, b1, b2, b3,
              g0, g1, g2, g3, w0, w1, w2, w3):
    bufs = (b0, b1, b2, b3)
    gsems = (g0, g1, g2, g3)
    wsems = (w0, w1, w2, w3)

    wid = lax.axis_index("s") * NC + lax.axis_index("c")
    base = wid * PER_W

    pltpu.sync_copy(idx_hbm.at[wid], idx_v)

    def gather(b, grp):
        return pltpu.make_async_copy(table_hbm.at[idx_v.at[grp]], bufs[b],
                                     gsems[b])

    def write(b, grp):
        return pltpu.make_async_copy(bufs[b],
                                     out_hbm.at[pl.ds(base + grp * G, G)],
                                     wsems[b])

    for b in range(NBUF):
        gather(b, b).start()

    @pl.loop(0, NG, step=NBUF)
    def _round(g):
        for b in range(NBUF):
            gather(b, g + b).wait()
            write(b, g + b).start()
        for b in range(NBUF):
            nxt = g + NBUF + b

            @pl.when(nxt < NG)
            def _():
                write(b, g + b).wait()
                gather(b, nxt).start()

    for b in range(NBUF):
        write(b, NG - NBUF + b).wait()


_emb = functools.partial(
    pl.kernel,
    out_type=jax.ShapeDtypeStruct((B_TOTAL, D), jnp.float32),
    mesh=plsc.VectorSubcoreMesh(core_axis_name="c", subcore_axis_name="s"),
    compiler_params=pltpu.CompilerParams(use_tc_tiling_on_sc=False),
    scratch_types=(
        [pltpu.VMEM((NG, G), jnp.int32)]
        + [pltpu.VMEM((G, D), jnp.float32) for _ in range(NBUF)]
        + [pltpu.SemaphoreType.DMA for _ in range(2 * NBUF)]
    ),
)(_emb_body)


@jax.jit
def kernel(x, weight):
    idx = x.reshape(NW, NG, G).astype(jnp.int32)
    out = _emb(idx, weight)
    return out.reshape(BATCH, HIST, D)


# speedup vs baseline: 1.8743x; 1.0032x over previous
import functools

import jax
import jax.numpy as jnp
from jax import lax
from jax.experimental import pallas as pl
from jax.experimental.pallas import tpu as pltpu
from jax.experimental.pallas import tpu_sc as plsc

NUM_EMBEDDINGS = 1000000
D = 64
BATCH = 16384
HIST = 50
B_TOTAL = BATCH * HIST

NC = 2
NS = 16
NW = NC * NS
G = 128
PER_W = B_TOTAL // NW
NG = PER_W // G
NBUF = 8


def _emb_body(idx_hbm, table_hbm, out_hbm, idx_v, *scratch):
    bufs = scratch[:NBUF]
    gsems = scratch[NBUF:2 * NBUF]
    wsems = scratch[2 * NBUF:]

    wid = lax.axis_index("s") * NC + lax.axis_index("c")
    base = wid * PER_W

    pltpu.sync_copy(idx_hbm.at[wid], idx_v)

    def gather(b, grp):
        return pltpu.make_async_copy(table_hbm.at[idx_v.at[grp]], bufs[b],
                                     gsems[b])

    def write(b, grp):
        return pltpu.make_async_copy(bufs[b],
                                     out_hbm.at[pl.ds(base + grp * G, G)],
                                     wsems[b])

    for b in range(NBUF):
        gather(b, b).start()

    @pl.loop(0, NG, step=NBUF)
    def _round(g):
        for b in range(NBUF):
            gather(b, g + b).wait()
            write(b, g + b).start()
        for b in range(NBUF):
            nxt = g + NBUF + b

            @pl.when(nxt < NG)
            def _():
                write(b, g + b).wait()
                gather(b, nxt).start()

    for b in range(NBUF):
        write(b, NG - NBUF + b).wait()


_emb = functools.partial(
    pl.kernel,
    out_type=jax.ShapeDtypeStruct((B_TOTAL, D), jnp.float32),
    mesh=plsc.VectorSubcoreMesh(core_axis_name="c", subcore_axis_name="s"),
    compiler_params=pltpu.CompilerParams(use_tc_tiling_on_sc=False),
    scratch_types=(
        [pltpu.VMEM((NG, G), jnp.int32)]
        + [pltpu.VMEM((G, D), jnp.float32) for _ in range(NBUF)]
        + [pltpu.SemaphoreType.DMA for _ in range(2 * NBUF)]
    ),
)(_emb_body)


@jax.jit
def kernel(x, weight):
    idx = x.reshape(NW, NG, G).astype(jnp.int32)
    out = _emb(idx, weight)
    return out.reshape(BATCH, HIST, D)


# speedup vs baseline: 1.8752x; 1.0005x over previous
import functools

import jax
import jax.numpy as jnp
from jax import lax
from jax.experimental import pallas as pl
from jax.experimental.pallas import tpu as pltpu
from jax.experimental.pallas import tpu_sc as plsc

NUM_EMBEDDINGS = 1000000
D = 64
BATCH = 16384
HIST = 50
B_TOTAL = BATCH * HIST

NC = 2
NS = 16
NW = NC * NS
G = 128
PER_W = B_TOTAL // NW
NG = PER_W // G
NBUF = 8


def _emb_body(idx_hbm, table_hbm, out_hbm, idx_v, *scratch):
    bufs = scratch[:NBUF]
    gsems = scratch[NBUF:2 * NBUF]
    wsems = scratch[2 * NBUF:]

    wid = lax.axis_index("s") * NC + lax.axis_index("c")
    base = wid * PER_W

    pltpu.sync_copy(idx_hbm.at[wid], idx_v)

    def gather(b, grp):
        return pltpu.make_async_copy(table_hbm.at[idx_v.at[grp]], bufs[b],
                                     gsems[b])

    def write(b, grp):
        return pltpu.make_async_copy(bufs[b],
                                     out_hbm.at[pl.ds(base + grp * G, G)],
                                     wsems[b])

    PF = NBUF // 2

    for m in range(PF):
        gather(m % NBUF, m).start()

    @pl.loop(0, NG, step=NBUF)
    def _round(g):
        for j in range(NBUF):
            m = g + j
            gather(j, m).wait()
            write(j, m).start()
            nxt = m + PF
            slot = (j + PF) % NBUF

            @pl.when(nxt < NG)
            def _():
                @pl.when(m >= PF)
                def _():
                    write(slot, m - PF).wait()

                gather(slot, nxt).start()

    for m in range(NG - 2 * PF, NG):
        write(m % NBUF, m).wait()


_emb = functools.partial(
    pl.kernel,
    out_type=jax.ShapeDtypeStruct((B_TOTAL, D), jnp.float32),
    mesh=plsc.VectorSubcoreMesh(core_axis_name="c", subcore_axis_name="s"),
    compiler_params=pltpu.CompilerParams(use_tc_tiling_on_sc=False),
    scratch_types=(
        [pltpu.VMEM((NG, G), jnp.int32)]
        + [pltpu.VMEM((G, D), jnp.float32) for _ in range(NBUF)]
        + [pltpu.SemaphoreType.DMA for _ in range(2 * NBUF)]
    ),
)(_emb_body)


@jax.jit
def kernel(x, weight):
    idx = x.reshape(NW, NG, G).astype(jnp.int32)
    out = _emb(idx, weight)
    return out.reshape(BATCH, HIST, D)
